# baseline (speedup 1.0000x reference)
import jax
import jax.numpy as jnp
from jax import lax
from jax.experimental import pallas as pl
from jax.experimental.pallas import tpu as pltpu
from jax.experimental.pallas import tpu_sc as plsc

N_NODES = 10000
N_EDGES = 320000
D = 128


_BM = 1000


def _mlp_body(x_ref, w1_ref, b1_ref, w2_ref, b2_ref, h_ref):
    h1 = jnp.dot(x_ref[...], w1_ref[...], preferred_element_type=jnp.float32)
    h1 = jnp.maximum(h1 + b1_ref[...], 0.0)
    h_ref[...] = (
        jnp.dot(h1, w2_ref[...], preferred_element_type=jnp.float32) + b2_ref[...]
    ).astype(jnp.bfloat16)


def _mlp(x, W1, b1, W2, b2):
    return pl.pallas_call(
        _mlp_body,
        grid=(N_NODES // _BM,),
        in_specs=[
            pl.BlockSpec((_BM, D), lambda i: (i, 0)),
            pl.BlockSpec((D, D), lambda i: (0, 0)),
            pl.BlockSpec((1, D), lambda i: (0, 0)),
            pl.BlockSpec((D, D), lambda i: (0, 0)),
            pl.BlockSpec((1, D), lambda i: (0, 0)),
        ],
        out_specs=pl.BlockSpec((_BM, D), lambda i: (i, 0)),
        out_shape=jax.ShapeDtypeStruct((N_NODES, D), jnp.bfloat16),
    )(x, W1, b1.reshape(1, D), W2, b2.reshape(1, D))


_NC = 2
_NS = 16
_NW = _NC * _NS
_EPW = N_EDGES // _NW
_C = 80
_NCHUNK = _EPW // _C
_G = _C // 16
_KU = 8


def _edge_body(h_hbm, src_hbm, dst_hbm, out_hbm,
               idx_u, idx_v, ru0, rv0, ru1, rv1, ru2, rv2, ru3, rv3,
               outb, sem0, sem1, sem2, sem3):
    wid = lax.axis_index("s") * _NC + lax.axis_index("c")
    base = wid * _EPW
    lanes = lax.iota(jnp.int32, 16)

    pltpu.sync_copy(src_hbm.at[wid], idx_u)
    pltpu.sync_copy(dst_hbm.at[wid], idx_v)

    def fire(g, ru, rv, sem):
        pltpu.async_copy(h_hbm.at[idx_u.at[g]], ru, sem)
        pltpu.async_copy(h_hbm.at[idx_v.at[g]], rv, sem)

    def drain(ru, rv, sem):
        pltpu.make_async_copy(h_hbm.at[idx_u.at[0]], ru, sem).wait()
        pltpu.make_async_copy(h_hbm.at[idx_v.at[0]], rv, sem).wait()

    def compute(g, ru, rv):
        def group(gi, carry2):
            res = jnp.zeros((16,), jnp.float32)
            for lane in range(16):
                e = gi * 16 + lane
                acc = jnp.zeros((16,), jnp.float32)
                for k32 in range(D // 32):
                    u = plsc.bitcast(ru[e, pl.ds(k32 * 16, 16)], jnp.bfloat16)
                    v = plsc.bitcast(rv[e, pl.ds(k32 * 16, 16)], jnp.bfloat16)
                    ua, ub = plsc.unpack(u, format=plsc.PackFormat.INTERLEAVED)
                    va, vb = plsc.unpack(v, format=plsc.PackFormat.INTERLEAVED)
                    acc = acc + ua * va + ub * vb
                res = jnp.where(lanes == lane, jnp.sum(acc), res)
            outb[pl.ds(g * _C + gi * 16, 16)] = res
            return carry2

        lax.fori_loop(0, _G, group, 0)

    bufs = [(ru0, rv0, sem0), (ru1, rv1, sem1),
            (ru2, rv2, sem2), (ru3, rv3, sem3)]
    fire(0, *bufs[0])
    fire(1, *bufs[1])
    fire(2, *bufs[2])

    def quad(p, carry):
        for b in range(4):
            g = p * 4 + b
            drain(*bufs[b])
            compute(g, bufs[b][0], bufs[b][1])

            @pl.when(g + 3 < _NCHUNK)
            def _():
                fire(g + 3, *bufs[(b + 3) % 4])

        return carry

    lax.fori_loop(0, (_NCHUNK - 1) // 4, quad, 0)
    drain(*bufs[0])
    compute(_NCHUNK - 1, bufs[0][0], bufs[0][1])

    pltpu.sync_copy(outb, out_hbm.at[pl.ds(base, _EPW)])


_edge_dot = pl.kernel(
    _edge_body,
    out_type=jax.ShapeDtypeStruct((N_EDGES,), jnp.float32),
    mesh=plsc.VectorSubcoreMesh(core_axis_name="c", subcore_axis_name="s"),
    compiler_params=pltpu.CompilerParams(
        needs_layout_passes=False, use_tc_tiling_on_sc=False
    ),
    scratch_types=[
        pltpu.VMEM((_NCHUNK, _C), jnp.int32),
        pltpu.VMEM((_NCHUNK, _C), jnp.int32),
        pltpu.VMEM((_C, D // 2), jnp.int32),
        pltpu.VMEM((_C, D // 2), jnp.int32),
        pltpu.VMEM((_C, D // 2), jnp.int32),
        pltpu.VMEM((_C, D // 2), jnp.int32),
        pltpu.VMEM((_C, D // 2), jnp.int32),
        pltpu.VMEM((_C, D // 2), jnp.int32),
        pltpu.VMEM((_C, D // 2), jnp.int32),
        pltpu.VMEM((_C, D // 2), jnp.int32),
        pltpu.VMEM((_EPW,), jnp.float32),
        pltpu.SemaphoreType.DMA,
        pltpu.SemaphoreType.DMA,
        pltpu.SemaphoreType.DMA,
        pltpu.SemaphoreType.DMA,
    ],
)


def kernel(x, edge_index, W1, b1, W2, b2):
    h = _mlp(x, W1, b1, W2, b2)
    h = lax.bitcast_convert_type(h.reshape(N_NODES, D // 2, 2), jnp.int32)
    src = edge_index[0].astype(jnp.int32).reshape(_NW, _NCHUNK, _C)
    dst = edge_index[1].astype(jnp.int32).reshape(_NW, _NCHUNK, _C)
    score = _edge_dot(h, src, dst)
    return score.reshape(N_EDGES, 1)


# speedup vs baseline: 1.2446x; 1.2446x over previous
import jax
import jax.numpy as jnp
from jax import lax
from jax.experimental import pallas as pl
from jax.experimental.pallas import tpu as pltpu
from jax.experimental.pallas import tpu_sc as plsc

N_NODES = 10000
N_EDGES = 320000
D = 128


_BM = 1000


def _mlp_body(x_ref, w1_ref, b1_ref, w2_ref, b2_ref, h_ref):
    h1 = jnp.dot(x_ref[...], w1_ref[...], preferred_element_type=jnp.float32)
    h1 = jnp.maximum(h1 + b1_ref[...], 0.0)
    h_ref[...] = (
        jnp.dot(h1, w2_ref[...], preferred_element_type=jnp.float32) + b2_ref[...]
    ).astype(jnp.bfloat16)


def _mlp(x, W1, b1, W2, b2):
    return pl.pallas_call(
        _mlp_body,
        grid=(N_NODES // _BM,),
        in_specs=[
            pl.BlockSpec((_BM, D), lambda i: (i, 0)),
            pl.BlockSpec((D, D), lambda i: (0, 0)),
            pl.BlockSpec((1, D), lambda i: (0, 0)),
            pl.BlockSpec((D, D), lambda i: (0, 0)),
            pl.BlockSpec((1, D), lambda i: (0, 0)),
        ],
        out_specs=pl.BlockSpec((_BM, D), lambda i: (i, 0)),
        out_shape=jax.ShapeDtypeStruct((N_NODES, D), jnp.bfloat16),
    )(x, W1, b1.reshape(1, D), W2, b2.reshape(1, D))


_NC = 2
_NS = 16
_NW = _NC * _NS
_EPW = N_EDGES // _NW
_C = 80
_NCHUNK = _EPW // _C
_G = _C // 16
_KU = 8


def _edge_body(h_hbm, src_hbm, dst_hbm, out_hbm,
               idx_u, idx_v, ru0, rv0, ru1, rv1, outb, sem0, sem1):
    wid = lax.axis_index("s") * _NC + lax.axis_index("c")
    base = wid * _EPW
    lanes = lax.iota(jnp.int32, 16)

    pltpu.sync_copy(src_hbm.at[wid], idx_u)
    pltpu.sync_copy(dst_hbm.at[wid], idx_v)

    def fire(g, ru, rv, sem):
        pltpu.async_copy(h_hbm.at[idx_u.at[g]], ru, sem)
        pltpu.async_copy(h_hbm.at[idx_v.at[g]], rv, sem)

    def drain(ru, rv, sem):
        pltpu.make_async_copy(h_hbm.at[idx_u.at[0]], ru, sem).wait()
        pltpu.make_async_copy(h_hbm.at[idx_v.at[0]], rv, sem).wait()

    def compute(g, ru, rv):
        def group(gi, carry2):
            res = jnp.zeros((16,), jnp.float32)
            for lane in range(16):
                e = gi * 16 + lane
                acc = jnp.zeros((16,), jnp.float32)
                for k32 in range(D // 32):
                    u = plsc.bitcast(ru[e, pl.ds(k32 * 16, 16)], jnp.bfloat16)
                    v = plsc.bitcast(rv[e, pl.ds(k32 * 16, 16)], jnp.bfloat16)
                    pa, pb = plsc.unpack(u * v, format=plsc.PackFormat.INTERLEAVED)
                    acc = acc + pa + pb
                res = jnp.where(lanes == lane, jnp.sum(acc), res)
            outb[pl.ds(g * _C + gi * 16, 16)] = res
            return carry2

        lax.fori_loop(0, _G, group, 0)

    fire(0, ru0, rv0, sem0)

    def pair(p, carry):
        g0 = p * 2
        fire(g0 + 1, ru1, rv1, sem1)
        drain(ru0, rv0, sem0)
        compute(g0, ru0, rv0)
        fire(g0 + 2, ru0, rv0, sem0)
        drain(ru1, rv1, sem1)
        compute(g0 + 1, ru1, rv1)
        return carry

    lax.fori_loop(0, (_NCHUNK - 1) // 2, pair, 0)
    drain(ru0, rv0, sem0)
    compute(_NCHUNK - 1, ru0, rv0)

    pltpu.sync_copy(outb, out_hbm.at[pl.ds(base, _EPW)])


_edge_dot = pl.kernel(
    _edge_body,
    out_type=jax.ShapeDtypeStruct((N_EDGES,), jnp.float32),
    mesh=plsc.VectorSubcoreMesh(core_axis_name="c", subcore_axis_name="s"),
    compiler_params=pltpu.CompilerParams(
        needs_layout_passes=False, use_tc_tiling_on_sc=False
    ),
    scratch_types=[
        pltpu.VMEM((_NCHUNK, _C), jnp.int32),
        pltpu.VMEM((_NCHUNK, _C), jnp.int32),
        pltpu.VMEM((_C, D // 2), jnp.int32),
        pltpu.VMEM((_C, D // 2), jnp.int32),
        pltpu.VMEM((_C, D // 2), jnp.int32),
        pltpu.VMEM((_C, D // 2), jnp.int32),
        pltpu.VMEM((_EPW,), jnp.float32),
        pltpu.SemaphoreType.DMA,
        pltpu.SemaphoreType.DMA,
    ],
)


def kernel(x, edge_index, W1, b1, W2, b2):
    h = _mlp(x, W1, b1, W2, b2)
    h = lax.bitcast_convert_type(h.reshape(N_NODES, D // 2, 2), jnp.int32)
    src = edge_index[0].astype(jnp.int32).reshape(_NW, _NCHUNK, _C)
    dst = edge_index[1].astype(jnp.int32).reshape(_NW, _NCHUNK, _C)
    score = _edge_dot(h, src, dst)
    return score.reshape(N_EDGES, 1)


# speedup vs baseline: 1.5056x; 1.2097x over previous
import jax
import jax.numpy as jnp
from jax import lax
from jax.experimental import pallas as pl
from jax.experimental.pallas import tpu as pltpu
from jax.experimental.pallas import tpu_sc as plsc

N_NODES = 10000
N_EDGES = 320000
D = 128


_BM = 1000


def _mlp_body(x_ref, w1_ref, b1_ref, w2_ref, b2_ref, h_ref):
    h1 = jnp.dot(x_ref[...], w1_ref[...], preferred_element_type=jnp.float32)
    h1 = jnp.maximum(h1 + b1_ref[...], 0.0)
    h_ref[...] = (
        jnp.dot(h1, w2_ref[...], preferred_element_type=jnp.float32) + b2_ref[...]
    ).astype(jnp.bfloat16)


def _mlp(x, W1, b1, W2, b2):
    return pl.pallas_call(
        _mlp_body,
        grid=(N_NODES // _BM,),
        in_specs=[
            pl.BlockSpec((_BM, D), lambda i: (i, 0)),
            pl.BlockSpec((D, D), lambda i: (0, 0)),
            pl.BlockSpec((1, D), lambda i: (0, 0)),
            pl.BlockSpec((D, D), lambda i: (0, 0)),
            pl.BlockSpec((1, D), lambda i: (0, 0)),
        ],
        out_specs=pl.BlockSpec((_BM, D), lambda i: (i, 0)),
        out_shape=jax.ShapeDtypeStruct((N_NODES, D), jnp.bfloat16),
    )(x, W1, b1.reshape(1, D), W2, b2.reshape(1, D))


_NC = 2
_NS = 16
_NW = _NC * _NS
_EPW = N_EDGES // _NW
_C = 80
_NCHUNK = _EPW // _C
_G = _C // 16
_KU = 8


def _edge_body(h_hbm, src_hbm, dst_hbm, out_hbm,
               idx_u, idx_v, ru0, rv0, ru1, rv1, outb, hsp, sem0, sem1):
    wid = lax.axis_index("s") * _NC + lax.axis_index("c")
    sid = lax.axis_index("s")
    base = wid * _EPW
    lanes = lax.iota(jnp.int32, 16)

    @pl.when(sid == 0)
    def _():
        pltpu.sync_copy(h_hbm, hsp)

    pltpu.sync_copy(src_hbm.at[wid], idx_u)
    pltpu.sync_copy(dst_hbm.at[wid], idx_v)
    plsc.subcore_barrier()

    def fire(g, ru, rv, sem):
        pltpu.async_copy(hsp.at[idx_u.at[g]], ru, sem)
        pltpu.async_copy(hsp.at[idx_v.at[g]], rv, sem)

    def drain(ru, rv, sem):
        pltpu.make_async_copy(hsp.at[idx_u.at[0]], ru, sem).wait()
        pltpu.make_async_copy(hsp.at[idx_v.at[0]], rv, sem).wait()

    def compute(g, ru, rv):
        def group(gi, carry2):
            res = jnp.zeros((16,), jnp.float32)
            for lane in range(16):
                e = gi * 16 + lane
                acc = jnp.zeros((16,), jnp.float32)
                for k32 in range(D // 32):
                    u = plsc.bitcast(ru[e, pl.ds(k32 * 16, 16)], jnp.bfloat16)
                    v = plsc.bitcast(rv[e, pl.ds(k32 * 16, 16)], jnp.bfloat16)
                    pa, pb = plsc.unpack(u * v, format=plsc.PackFormat.INTERLEAVED)
                    acc = acc + pa + pb
                res = jnp.where(lanes == lane, jnp.sum(acc), res)
            outb[pl.ds(g * _C + gi * 16, 16)] = res
            return carry2

        lax.fori_loop(0, _G, group, 0)

    fire(0, ru0, rv0, sem0)

    def pair(p, carry):
        g0 = p * 2
        fire(g0 + 1, ru1, rv1, sem1)
        drain(ru0, rv0, sem0)
        compute(g0, ru0, rv0)
        fire(g0 + 2, ru0, rv0, sem0)
        drain(ru1, rv1, sem1)
        compute(g0 + 1, ru1, rv1)
        return carry

    lax.fori_loop(0, (_NCHUNK - 1) // 2, pair, 0)
    drain(ru0, rv0, sem0)
    compute(_NCHUNK - 1, ru0, rv0)

    pltpu.sync_copy(outb, out_hbm.at[pl.ds(base, _EPW)])


_edge_dot = pl.kernel(
    _edge_body,
    out_type=jax.ShapeDtypeStruct((N_EDGES,), jnp.float32),
    mesh=plsc.VectorSubcoreMesh(core_axis_name="c", subcore_axis_name="s"),
    compiler_params=pltpu.CompilerParams(
        needs_layout_passes=False, use_tc_tiling_on_sc=False
    ),
    scratch_types=[
        pltpu.VMEM((_NCHUNK, _C), jnp.int32),
        pltpu.VMEM((_NCHUNK, _C), jnp.int32),
        pltpu.VMEM((_C, D // 2), jnp.int32),
        pltpu.VMEM((_C, D // 2), jnp.int32),
        pltpu.VMEM((_C, D // 2), jnp.int32),
        pltpu.VMEM((_C, D // 2), jnp.int32),
        pltpu.VMEM((_EPW,), jnp.float32),
        pltpu.VMEM_SHARED((N_NODES, D // 2), jnp.int32),
        pltpu.SemaphoreType.DMA,
        pltpu.SemaphoreType.DMA,
    ],
)


def kernel(x, edge_index, W1, b1, W2, b2):
    h = _mlp(x, W1, b1, W2, b2)
    h = lax.bitcast_convert_type(h.reshape(N_NODES, D // 2, 2), jnp.int32)
    src = edge_index[0].astype(jnp.int32).reshape(_NW, _NCHUNK, _C)
    dst = edge_index[1].astype(jnp.int32).reshape(_NW, _NCHUNK, _C)
    score = _edge_dot(h, src, dst)
    return score.reshape(N_EDGES, 1)
